# baseline (speedup 1.0000x reference)
import numpy as np
import jax
import jax.numpy as jnp
from jax.experimental import pallas as pl
from jax.sharding import Mesh, PartitionSpec as P

_NUM_EMBED = 1024
_EMBED_DIM = 64
_COMMIT = 0.25


def _vq_kernel(z_ref, w_ref, zq_ref, idx_ref, sse_ref):
    z = z_ref[0]
    w = w_ref[...]
    wsq = jnp.sum(w * w, axis=1, keepdims=True)
    zsq = jnp.sum(z * z, axis=0, keepdims=True)
    mm = jax.lax.dot_general(-2.0 * w, z, (((1,), (0,)), ((), ())),
                             preferred_element_type=jnp.float32)
    scores = (zsq + wsq) + mm
    minv = jnp.min(scores, axis=0, keepdims=True)
    cio = jax.lax.broadcasted_iota(jnp.int32, scores.shape, 0)
    idx = jnp.min(jnp.where(scores == minv, cio, jnp.int32(2**30)), axis=0)
    idx_ref[0, 0, :] = idx
    onehot = (cio == idx[None, :]).astype(jnp.float32)
    zq = jax.lax.dot_general(w, onehot, (((0,), (0,)), ((), ())),
                             preferred_element_type=jnp.float32)
    zq_ref[0] = zq
    sse_ref[0] = jnp.full((8, 128), jnp.sum((zq - z) ** 2), jnp.float32)


def _vq_block(z4, W):
    B, D, H, Wd = z4.shape
    T = H * Wd
    z3 = z4.reshape(B, D, T)
    zq3, idx3, sse = pl.pallas_call(
        _vq_kernel,
        grid=(B,),
        in_specs=[
            pl.BlockSpec((1, D, T), lambda b: (b, 0, 0)),
            pl.BlockSpec((_NUM_EMBED, D), lambda b: (0, 0)),
        ],
        out_specs=[
            pl.BlockSpec((1, D, T), lambda b: (b, 0, 0)),
            pl.BlockSpec((1, 1, T), lambda b: (b, 0, 0)),
            pl.BlockSpec((1, 8, 128), lambda b: (b, 0, 0)),
        ],
        out_shape=[
            jax.ShapeDtypeStruct((B, D, T), jnp.float32),
            jax.ShapeDtypeStruct((B, 1, T), jnp.int32),
            jax.ShapeDtypeStruct((B, 8, 128), jnp.float32),
        ],
    )(z3, W)
    return zq3.reshape(B, D, H, Wd), idx3.reshape(B, H, Wd), sse


def kernel(z_e, W):
    B, D, H, Wd = z_e.shape
    devs = jax.devices()
    nd = 1
    for cand in (8, 4, 2):
        if len(devs) >= cand and B % cand == 0:
            nd = cand
            break
    if nd > 1:
        mesh = Mesh(np.array(devs[:nd]), ("x",))
        fn = jax.shard_map(
            _vq_block, mesh=mesh,
            in_specs=(P("x"), P()),
            out_specs=(P("x"), P("x"), P("x")),
            check_vma=False,
        )
        zq4, indices, sse = fn(z_e, W)
    else:
        zq4, indices, sse = _vq_block(z_e, W)
    vq_loss = jnp.sum(sse[:, 0, 0]) / jnp.float32(B * D * H * Wd)
    commitment_loss = jnp.float32(_COMMIT) * vq_loss
    return (zq4, indices, vq_loss, commitment_loss)


# speedup vs baseline: 7.6557x; 7.6557x over previous
import jax
import jax.numpy as jnp
from jax.experimental import pallas as pl
from jax.experimental.pallas import tpu as pltpu

_NUM_EMBED = 1024
_EMBED_DIM = 64
_COMMIT = 0.25


def _vq_kernel(z_ref, w_ref, zq_ref, idx_ref, sse_ref, wn_ref, wsq_ref):
    b = pl.program_id(0)

    @pl.when(b == 0)
    def _prep():
        w0 = w_ref[...]
        wn_ref[...] = -2.0 * w0
        wsq_ref[...] = jnp.sum(w0 * w0, axis=1, keepdims=True)

    z = z_ref[0]
    w = w_ref[...]
    wsq = wsq_ref[...]
    zsq = jnp.sum(z * z, axis=0, keepdims=True)
    mm = jax.lax.dot_general(wn_ref[...], z, (((1,), (0,)), ((), ())),
                             preferred_element_type=jnp.float32)
    scores = (zsq + wsq) + mm
    minv = jnp.min(scores, axis=0, keepdims=True)
    cio = jax.lax.broadcasted_iota(jnp.int32, scores.shape, 0)
    idx = jnp.min(jnp.where(scores == minv, cio, jnp.int32(2**30)), axis=0)
    idx_ref[0, 0, :] = idx
    onehot = (cio == idx[None, :]).astype(jnp.float32)
    zq = jax.lax.dot_general(w, onehot, (((0,), (0,)), ((), ())),
                             preferred_element_type=jnp.float32)
    zq_ref[0] = zq
    sse = jnp.full((8, 128), jnp.sum((zq - z) ** 2), jnp.float32)

    @pl.when(b == 0)
    def _init():
        sse_ref[...] = sse

    @pl.when(b > 0)
    def _acc():
        sse_ref[...] = sse_ref[...] + sse


def kernel(z_e, W):
    B, D, H, Wd = z_e.shape
    T = H * Wd
    z3 = z_e.reshape(B, D, T)
    zq3, idx3, sse = pl.pallas_call(
        _vq_kernel,
        grid=(B,),
        in_specs=[
            pl.BlockSpec((1, D, T), lambda b: (b, 0, 0)),
            pl.BlockSpec((_NUM_EMBED, D), lambda b: (0, 0)),
        ],
        out_specs=[
            pl.BlockSpec((1, D, T), lambda b: (b, 0, 0)),
            pl.BlockSpec((1, 1, T), lambda b: (b, 0, 0)),
            pl.BlockSpec((8, 128), lambda b: (0, 0)),
        ],
        out_shape=[
            jax.ShapeDtypeStruct((B, D, T), jnp.float32),
            jax.ShapeDtypeStruct((B, 1, T), jnp.int32),
            jax.ShapeDtypeStruct((8, 128), jnp.float32),
        ],
        scratch_shapes=[
            pltpu.VMEM((_NUM_EMBED, _EMBED_DIM), jnp.float32),
            pltpu.VMEM((_NUM_EMBED, 1), jnp.float32),
        ],
    )(z3, W)
    z_q_st = zq3.reshape(B, D, H, Wd)
    indices = idx3.reshape(B, H, Wd)
    vq_loss = sse[0, 0] / jnp.float32(B * D * T)
    commitment_loss = jnp.float32(_COMMIT) * vq_loss
    return (z_q_st, indices, vq_loss, commitment_loss)


# speedup vs baseline: 7.7844x; 1.0168x over previous
import jax
import jax.numpy as jnp
from jax.experimental import pallas as pl
from jax.experimental.pallas import tpu as pltpu

_NUM_EMBED = 1024
_EMBED_DIM = 64
_COMMIT = 0.25


def _vq_kernel(z_ref, w_ref, zq_ref, idx_ref, sse_ref):
    z = z_ref[0]
    w = w_ref[...]
    wsq = jnp.sum(w * w, axis=1, keepdims=True)
    zsq = jnp.sum(z * z, axis=0, keepdims=True)
    mm = jax.lax.dot_general(-2.0 * w, z, (((1,), (0,)), ((), ())),
                             preferred_element_type=jnp.float32)
    scores = (zsq + wsq) + mm
    minv = jnp.min(scores, axis=0, keepdims=True)
    cio = jax.lax.broadcasted_iota(jnp.int32, scores.shape, 0)
    idx = jnp.min(jnp.where(scores == minv, cio, jnp.int32(2**30)), axis=0)
    idx_ref[0, 0, :] = idx
    onehot = (cio == idx[None, :]).astype(jnp.float32)
    zq = jax.lax.dot_general(w, onehot, (((0,), (0,)), ((), ())),
                             preferred_element_type=jnp.float32)
    zq_ref[0] = zq
    sse_ref[0] = jnp.full((8, 128), jnp.sum((zq - z) ** 2), jnp.float32)


def kernel(z_e, W):
    B, D, H, Wd = z_e.shape
    T = H * Wd
    z3 = z_e.reshape(B, D, T)
    zq3, idx3, sse = pl.pallas_call(
        _vq_kernel,
        grid=(B,),
        in_specs=[
            pl.BlockSpec((1, D, T), lambda b: (b, 0, 0)),
            pl.BlockSpec((_NUM_EMBED, D), lambda b: (0, 0)),
        ],
        out_specs=[
            pl.BlockSpec((1, D, T), lambda b: (b, 0, 0)),
            pl.BlockSpec((1, 1, T), lambda b: (b, 0, 0)),
            pl.BlockSpec((1, 8, 128), lambda b: (b, 0, 0)),
        ],
        out_shape=[
            jax.ShapeDtypeStruct((B, D, T), jnp.float32),
            jax.ShapeDtypeStruct((B, 1, T), jnp.int32),
            jax.ShapeDtypeStruct((B, 8, 128), jnp.float32),
        ],
        compiler_params=pltpu.CompilerParams(
            dimension_semantics=("parallel",),
        ),
    )(z3, W)
    z_q_st = zq3.reshape(B, D, H, Wd)
    indices = idx3.reshape(B, H, Wd)
    vq_loss = jnp.sum(sse[:, 0, 0]) / jnp.float32(B * D * T)
    commitment_loss = jnp.float32(_COMMIT) * vq_loss
    return (z_q_st, indices, vq_loss, commitment_loss)


# speedup vs baseline: 8.0984x; 1.0403x over previous
import jax
import jax.numpy as jnp
from jax.experimental import pallas as pl
from jax.experimental.pallas import tpu as pltpu

_NUM_EMBED = 1024
_EMBED_DIM = 64
_COMMIT = 0.25


def _vq_kernel(z_ref, w_ref, zq_ref, idx_ref, sse_ref):
    w = w_ref[...]
    wsq = jnp.sum(w * w, axis=1, keepdims=True)
    wn = -2.0 * w
    sse = jnp.float32(0.0)
    for j in range(2):
        z = z_ref[j]
        zsq = jnp.sum(z * z, axis=0, keepdims=True)
        mm = jax.lax.dot_general(wn, z, (((1,), (0,)), ((), ())),
                                 preferred_element_type=jnp.float32)
        scores = (zsq + wsq) + mm
        minv = jnp.min(scores, axis=0, keepdims=True)
        cio = jax.lax.broadcasted_iota(jnp.int32, scores.shape, 0)
        idx = jnp.min(jnp.where(scores == minv, cio, jnp.int32(2**30)), axis=0)
        idx_ref[j, 0, :] = idx
        onehot = (cio == idx[None, :]).astype(jnp.float32)
        zq = jax.lax.dot_general(w, onehot, (((0,), (0,)), ((), ())),
                                 preferred_element_type=jnp.float32)
        zq_ref[j] = zq
        sse = sse + jnp.sum((zq - z) ** 2)
    sse_ref[0] = jnp.full((8, 128), sse, jnp.float32)


def kernel(z_e, W):
    B, D, H, Wd = z_e.shape
    T = H * Wd
    z3 = z_e.reshape(B, D, T)
    zq3, idx3, sse = pl.pallas_call(
        _vq_kernel,
        grid=(B // 2,),
        in_specs=[
            pl.BlockSpec((2, D, T), lambda b: (b, 0, 0)),
            pl.BlockSpec((_NUM_EMBED, D), lambda b: (0, 0)),
        ],
        out_specs=[
            pl.BlockSpec((2, D, T), lambda b: (b, 0, 0)),
            pl.BlockSpec((2, 1, T), lambda b: (b, 0, 0)),
            pl.BlockSpec((1, 8, 128), lambda b: (b, 0, 0)),
        ],
        out_shape=[
            jax.ShapeDtypeStruct((B, D, T), jnp.float32),
            jax.ShapeDtypeStruct((B, 1, T), jnp.int32),
            jax.ShapeDtypeStruct((B // 2, 8, 128), jnp.float32),
        ],
        compiler_params=pltpu.CompilerParams(
            dimension_semantics=("parallel",),
        ),
    )(z3, W)
    z_q_st = zq3.reshape(B, D, H, Wd)
    indices = idx3.reshape(B, H, Wd)
    vq_loss = jnp.sum(sse[:, 0, 0]) / jnp.float32(B * D * T)
    commitment_loss = jnp.float32(_COMMIT) * vq_loss
    return (z_q_st, indices, vq_loss, commitment_loss)


# speedup vs baseline: 8.1704x; 1.0089x over previous
import jax
import jax.numpy as jnp
from jax.experimental import pallas as pl
from jax.experimental.pallas import tpu as pltpu

_NUM_EMBED = 1024
_EMBED_DIM = 64
_COMMIT = 0.25


def _vq_kernel(z_ref, w_ref, zq_ref, idx_ref, sse_ref):
    w = w_ref[...]
    wsq = jnp.sum(w * w, axis=1, keepdims=True)
    wn = -2.0 * w
    sse = jnp.float32(0.0)
    for j in range(4):
        z = z_ref[j]
        zsq = jnp.sum(z * z, axis=0, keepdims=True)
        mm = jax.lax.dot_general(wn, z, (((1,), (0,)), ((), ())),
                                 preferred_element_type=jnp.float32)
        scores = (zsq + wsq) + mm
        minv = jnp.min(scores, axis=0, keepdims=True)
        cio = jax.lax.broadcasted_iota(jnp.int32, scores.shape, 0)
        idx = jnp.min(jnp.where(scores == minv, cio, jnp.int32(2**30)), axis=0)
        idx_ref[j, 0, :] = idx
        onehot = (cio == idx[None, :]).astype(jnp.float32)
        zq = jax.lax.dot_general(w, onehot, (((0,), (0,)), ((), ())),
                                 preferred_element_type=jnp.float32)
        zq_ref[j] = zq
        sse = sse + jnp.sum((zq - z) ** 2)
    sse_ref[0] = jnp.full((8, 128), sse, jnp.float32)


def kernel(z_e, W):
    B, D, H, Wd = z_e.shape
    T = H * Wd
    z3 = z_e.reshape(B, D, T)
    zq3, idx3, sse = pl.pallas_call(
        _vq_kernel,
        grid=(B // 4,),
        in_specs=[
            pl.BlockSpec((4, D, T), lambda b: (b, 0, 0)),
            pl.BlockSpec((_NUM_EMBED, D), lambda b: (0, 0)),
        ],
        out_specs=[
            pl.BlockSpec((4, D, T), lambda b: (b, 0, 0)),
            pl.BlockSpec((4, 1, T), lambda b: (b, 0, 0)),
            pl.BlockSpec((1, 8, 128), lambda b: (b, 0, 0)),
        ],
        out_shape=[
            jax.ShapeDtypeStruct((B, D, T), jnp.float32),
            jax.ShapeDtypeStruct((B, 1, T), jnp.int32),
            jax.ShapeDtypeStruct((B // 4, 8, 128), jnp.float32),
        ],
        compiler_params=pltpu.CompilerParams(
            dimension_semantics=("parallel",),
        ),
    )(z3, W)
    z_q_st = zq3.reshape(B, D, H, Wd)
    indices = idx3.reshape(B, H, Wd)
    vq_loss = jnp.sum(sse[:, 0, 0]) / jnp.float32(B * D * T)
    commitment_loss = jnp.float32(_COMMIT) * vq_loss
    return (z_q_st, indices, vq_loss, commitment_loss)


# speedup vs baseline: 8.1722x; 1.0002x over previous
import jax
import jax.numpy as jnp
from jax.experimental import pallas as pl
from jax.experimental.pallas import tpu as pltpu

_NUM_EMBED = 1024
_EMBED_DIM = 64
_COMMIT = 0.25


def _vq_kernel(z_ref, w_ref, zq_ref, idx_ref, sse_ref):
    w = w_ref[...]
    wsq = jnp.sum(w * w, axis=1, keepdims=True)
    wn = -2.0 * w
    sse = jnp.float32(0.0)
    for j in range(8):
        z = z_ref[j]
        zsq = jnp.sum(z * z, axis=0, keepdims=True)
        mm = jax.lax.dot_general(wn, z, (((1,), (0,)), ((), ())),
                                 preferred_element_type=jnp.float32)
        scores = (zsq + wsq) + mm
        minv = jnp.min(scores, axis=0, keepdims=True)
        cio = jax.lax.broadcasted_iota(jnp.int32, scores.shape, 0)
        idx = jnp.min(jnp.where(scores == minv, cio, jnp.int32(2**30)), axis=0)
        idx_ref[j, 0, :] = idx
        onehot = (cio == idx[None, :]).astype(jnp.float32)
        zq = jax.lax.dot_general(w, onehot, (((0,), (0,)), ((), ())),
                                 preferred_element_type=jnp.float32)
        zq_ref[j] = zq
        sse = sse + jnp.sum((zq - z) ** 2)
    sse_ref[0] = jnp.full((8, 128), sse, jnp.float32)


def kernel(z_e, W):
    B, D, H, Wd = z_e.shape
    T = H * Wd
    z3 = z_e.reshape(B, D, T)
    zq3, idx3, sse = pl.pallas_call(
        _vq_kernel,
        grid=(B // 8,),
        in_specs=[
            pl.BlockSpec((8, D, T), lambda b: (b, 0, 0)),
            pl.BlockSpec((_NUM_EMBED, D), lambda b: (0, 0)),
        ],
        out_specs=[
            pl.BlockSpec((8, D, T), lambda b: (b, 0, 0)),
            pl.BlockSpec((8, 1, T), lambda b: (b, 0, 0)),
            pl.BlockSpec((1, 8, 128), lambda b: (b, 0, 0)),
        ],
        out_shape=[
            jax.ShapeDtypeStruct((B, D, T), jnp.float32),
            jax.ShapeDtypeStruct((B, 1, T), jnp.int32),
            jax.ShapeDtypeStruct((B // 8, 8, 128), jnp.float32),
        ],
        compiler_params=pltpu.CompilerParams(
            dimension_semantics=("parallel",),
        ),
    )(z3, W)
    z_q_st = zq3.reshape(B, D, H, Wd)
    indices = idx3.reshape(B, H, Wd)
    vq_loss = jnp.sum(sse[:, 0, 0]) / jnp.float32(B * D * T)
    commitment_loss = jnp.float32(_COMMIT) * vq_loss
    return (z_q_st, indices, vq_loss, commitment_loss)
